# baseline (speedup 1.0000x reference)
import functools

import jax
import jax.numpy as jnp
from jax import lax
from jax.experimental import pallas as pl
from jax.experimental.pallas import tpu as pltpu
from jax.experimental.pallas import tpu_sc as plsc

_ROWS = 256
_COLS = 256
_NEDGE = 64
_NSAM = 64
_NC = 2
_NS = 16
_NW = _NC * _NS
_RG = 16
_TROWS = 64
_NTASK = _NSAM * (_ROWS // _TROWS)
_TPW = _NTASK // _NW


def _raster_sc(pts_hbm, zeros_hbm, hist_hbm, pts_v, hist_v, sems):
    wid = lax.axis_index("s") * _NC + lax.axis_index("c")
    lane = lax.broadcasted_iota(jnp.int32, (_RG,), 0)
    lane_f = lane.astype(jnp.float32)
    ones = jnp.ones((_RG,), jnp.float32)
    n_rg = _TROWS // _RG

    def task(t, b, sem):
        g = wid * _TPW + t
        sample = g // (_ROWS // _TROWS)
        rg = g - sample * (_ROWS // _TROWS)

        @pl.when(t >= 2)
        def _drain():
            pltpu.make_async_copy(
                hist_v.at[b], hist_hbm.at[sample, pl.ds(rg * _TROWS, _TROWS)],
                sem).wait()

        @pl.when(rg == 0)
        def _load_pts():
            pltpu.sync_copy(pts_hbm.at[sample], pts_v)

        pltpu.sync_copy(zeros_hbm, hist_v.at[b])

        chunks = [jnp.clip(pts_v[pl.ds(c * _RG, _RG)] * 255.0, 0.0, 255.0)
                  for c in range(4 * _NEDGE // _RG)]

        ys0 = (rg * _TROWS).astype(jnp.float32)
        yss = [lane_f + (ys0 + float(r * _RG)) for r in range(n_rg)]
        rows = [lane + r * _RG for r in range(n_rg)]
        for e in range(_NEDGE):
            c, j = e // _RG, e % _RG
            idx_j = jnp.full((_RG,), j, jnp.int32)
            px = chunks[c].at[idx_j].get(mode="promise_in_bounds")
            py = chunks[4 + c].at[idx_j].get(mode="promise_in_bounds")
            pjx = chunks[8 + c].at[idx_j].get(mode="promise_in_bounds")
            pjy = chunks[12 + c].at[idx_j].get(mode="promise_in_bounds")
            dy = pjy - py
            denom = jnp.where(dy == 0.0, 1.0, dy)
            dx = pjx - px
            for r in range(n_rg):
                ys = yss[r]
                cond = ((py < ys) & (pjy >= ys)) | ((pjy < ys) & (py >= ys))
                q = (ys - py) / denom
                xc = px + q * dx
                bins = jnp.clip(xc, 0.0, 255.0).astype(jnp.int32)
                plsc.addupdate_scatter(hist_v.at[b], [rows[r], bins], ones,
                                       mask=cond)

        pltpu.async_copy(
            hist_v.at[b], hist_hbm.at[sample, pl.ds(rg * _TROWS, _TROWS)], sem)

    def pair(i, carry):
        task(i * 2, 0, sems.at[0])
        task(i * 2 + 1, 1, sems.at[1])
        return carry

    lax.fori_loop(0, _TPW // 2, pair, 0)

    for b in range(2):
        pltpu.make_async_copy(hist_v.at[b], hist_hbm.at[0, pl.ds(0, _TROWS)],
                              sems.at[b]).wait()


def _dice_tc(hist_ref, dmap_ref, out_ref):
    s = pl.program_id(0)

    jj = jax.lax.broadcasted_iota(jnp.int32, (_COLS, _COLS), 0)
    xx = jax.lax.broadcasted_iota(jnp.int32, (_COLS, _COLS), 1)
    l_incl = (jj <= xx).astype(jnp.float32)

    hist = hist_ref[0]
    a = jax.lax.dot(hist, l_incl, preferred_element_type=jnp.float32)
    b = a - hist
    r_tot = a[:, _COLS - 1:_COLS]
    m_lim = r_tot - 1.0 - (r_tot - 2.0 * jnp.floor(r_tot * 0.5))
    b_odd = b - 2.0 * jnp.floor(b * 0.5)
    f_odd = (b <= m_lim).astype(jnp.float32)
    f_even = ((a > b).astype(jnp.float32)
              * ((b + 1.0) <= m_lim).astype(jnp.float32))
    filled = b_odd * f_odd + (1.0 - b_odd) * f_even

    binary = (dmap_ref[0] * 255.0 <= 127.0).astype(jnp.float32)

    inter = jnp.sum(filled * binary)
    s_true = jnp.sum(filled)
    s_pred = jnp.sum(binary)

    smooth = 1e-06
    loss = 1.0 - (2.0 * inter + smooth) / (s_true + s_pred + smooth)

    @pl.when(s == 0)
    def _init():
        out_ref[...] = jnp.zeros_like(out_ref)

    out_ref[...] += loss * (1.0 / _NSAM)


@jax.jit
def _run(pts_sc, dmap):
    zeros = jnp.zeros((_TROWS, _COLS), jnp.float32)
    raster = pl.kernel(
        _raster_sc,
        out_type=jax.ShapeDtypeStruct((_NSAM, _ROWS, _COLS), jnp.float32),
        mesh=plsc.VectorSubcoreMesh(core_axis_name="c", subcore_axis_name="s",
                                    num_cores=_NC, num_subcores=_NS),
        scratch_types=[
            pltpu.VMEM((4 * _NEDGE,), jnp.float32),
            pltpu.VMEM((2, _TROWS, _COLS), jnp.float32),
            pltpu.SemaphoreType.DMA((2,)),
        ],
        compiler_params=pltpu.CompilerParams(use_tc_tiling_on_sc=False,
                                             needs_layout_passes=False),
    )
    hist = raster(pts_sc, zeros)

    out = pl.pallas_call(
        _dice_tc,
        grid=(_NSAM,),
        in_specs=[
            pl.BlockSpec((1, _ROWS, _COLS), lambda s: (s, 0, 0)),
            pl.BlockSpec((1, _ROWS, _COLS), lambda s: (s, 0, 0)),
        ],
        out_specs=pl.BlockSpec((8, 128), lambda s: (0, 0)),
        out_shape=jax.ShapeDtypeStruct((8, 128), jnp.float32),
        compiler_params=pltpu.CompilerParams(
            dimension_semantics=("arbitrary",),
        ),
    )(hist, dmap)
    return out[0, 0]


def kernel(points, distance_map):
    pts = points[:, :, 0, :]
    px = pts[:, :, 0]
    py = pts[:, :, 1]
    pjx = jnp.roll(px, 1, axis=1)
    pjy = jnp.roll(py, 1, axis=1)
    pts_sc = jnp.concatenate([px, py, pjx, pjy], axis=1)
    dmap = distance_map[:, :, :, 0]
    return _run(pts_sc, dmap)


# speedup vs baseline: 1.0773x; 1.0773x over previous
import functools

import jax
import jax.numpy as jnp
from jax import lax
from jax.experimental import pallas as pl
from jax.experimental.pallas import tpu as pltpu
from jax.experimental.pallas import tpu_sc as plsc

_ROWS = 256
_COLS = 256
_NEDGE = 64
_NSAM = 64
_NC = 2
_NS = 16
_NW = _NC * _NS
_RG = 16
_TROWS = 64
_NTASK = _NSAM * (_ROWS // _TROWS)
_TPW = _NTASK // _NW


def _raster_sc(pts_hbm, zeros_hbm, hist_hbm, pts_v, hist_v, sems):
    wid = lax.axis_index("s") * _NC + lax.axis_index("c")
    lane = lax.broadcasted_iota(jnp.int32, (_RG,), 0)
    lane_f = lane.astype(jnp.float32)
    ones = jnp.ones((_RG,), jnp.float32)
    n_rg = _TROWS // _RG

    def task(t, b, sem):
        g = wid * _TPW + t
        sample = g // (_ROWS // _TROWS)
        rg = g - sample * (_ROWS // _TROWS)

        @pl.when(t >= 2)
        def _drain():
            pltpu.make_async_copy(
                hist_v.at[b, :, pl.ds(0, _COLS)],
                hist_hbm.at[sample, pl.ds(rg * _TROWS, _TROWS)],
                sem).wait()

        @pl.when(rg == 0)
        def _load_pts():
            pltpu.sync_copy(pts_hbm.at[sample], pts_v)

        pltpu.sync_copy(zeros_hbm, hist_v.at[b])

        chunks = [jnp.clip(pts_v[pl.ds(c * _RG, _RG)] * 255.0, 0.0, 255.0)
                  for c in range(4 * _NEDGE // _RG)]

        ys0 = (rg * _TROWS).astype(jnp.float32)
        yss = [lane_f + (ys0 + float(r * _RG)) for r in range(n_rg)]
        rows = [lane + r * _RG for r in range(n_rg)]
        for e in range(_NEDGE):
            c, j = e // _RG, e % _RG
            idx_j = jnp.full((_RG,), j, jnp.int32)
            px = chunks[c].at[idx_j].get(mode="promise_in_bounds")
            py = chunks[4 + c].at[idx_j].get(mode="promise_in_bounds")
            pjx = chunks[8 + c].at[idx_j].get(mode="promise_in_bounds")
            pjy = chunks[12 + c].at[idx_j].get(mode="promise_in_bounds")
            dy = pjy - py
            denom = jnp.where(dy == 0.0, 1.0, dy)
            dx = pjx - px
            for r in range(n_rg):
                ys = yss[r]
                cond = ((py < ys) & (pjy >= ys)) | ((pjy < ys) & (py >= ys))
                q = (ys - py) / denom
                xc = px + q * dx
                bins = jnp.clip(xc, 0.0, 255.0).astype(jnp.int32)
                plsc.addupdate_scatter(hist_v.at[b], [rows[r], bins], ones,
                                       mask=cond)

        pltpu.async_copy(
            hist_v.at[b, :, pl.ds(0, _COLS)],
            hist_hbm.at[sample, pl.ds(rg * _TROWS, _TROWS)], sem)

    def pair(i, carry):
        task(i * 2, 0, sems.at[0])
        task(i * 2 + 1, 1, sems.at[1])
        return carry

    lax.fori_loop(0, _TPW // 2, pair, 0)

    for b in range(2):
        pltpu.make_async_copy(hist_v.at[b, :, pl.ds(0, _COLS)],
                              hist_hbm.at[0, pl.ds(0, _TROWS)],
                              sems.at[b]).wait()


def _dice_tc(hist_ref, dmap_ref, out_ref):
    s = pl.program_id(0)

    jj = jax.lax.broadcasted_iota(jnp.int32, (_COLS, _COLS), 0)
    xx = jax.lax.broadcasted_iota(jnp.int32, (_COLS, _COLS), 1)
    l_incl = (jj <= xx).astype(jnp.float32)

    hist = hist_ref[0]
    a = jax.lax.dot(hist, l_incl, preferred_element_type=jnp.float32)
    b = a - hist
    r_tot = a[:, _COLS - 1:_COLS]
    m_lim = r_tot - 1.0 - (r_tot - 2.0 * jnp.floor(r_tot * 0.5))
    b_odd = b - 2.0 * jnp.floor(b * 0.5)
    f_odd = (b <= m_lim).astype(jnp.float32)
    f_even = ((a > b).astype(jnp.float32)
              * ((b + 1.0) <= m_lim).astype(jnp.float32))
    filled = b_odd * f_odd + (1.0 - b_odd) * f_even

    binary = (dmap_ref[0] * 255.0 <= 127.0).astype(jnp.float32)

    inter = jnp.sum(filled * binary)
    s_true = jnp.sum(filled)
    s_pred = jnp.sum(binary)

    smooth = 1e-06
    loss = 1.0 - (2.0 * inter + smooth) / (s_true + s_pred + smooth)

    @pl.when(s == 0)
    def _init():
        out_ref[...] = jnp.zeros_like(out_ref)

    out_ref[...] += loss * (1.0 / _NSAM)


@jax.jit
def _run(pts_sc, dmap):
    zeros = jnp.zeros((_TROWS, _COLS + 1), jnp.float32)
    raster = pl.kernel(
        _raster_sc,
        out_type=jax.ShapeDtypeStruct((_NSAM, _ROWS, _COLS), jnp.float32),
        mesh=plsc.VectorSubcoreMesh(core_axis_name="c", subcore_axis_name="s",
                                    num_cores=_NC, num_subcores=_NS),
        scratch_types=[
            pltpu.VMEM((4 * _NEDGE,), jnp.float32),
            pltpu.VMEM((2, _TROWS, _COLS + 1), jnp.float32),
            pltpu.SemaphoreType.DMA((2,)),
        ],
        compiler_params=pltpu.CompilerParams(use_tc_tiling_on_sc=False,
                                             needs_layout_passes=False),
    )
    hist = raster(pts_sc, zeros)

    out = pl.pallas_call(
        _dice_tc,
        grid=(_NSAM,),
        in_specs=[
            pl.BlockSpec((1, _ROWS, _COLS), lambda s: (s, 0, 0)),
            pl.BlockSpec((1, _ROWS, _COLS), lambda s: (s, 0, 0)),
        ],
        out_specs=pl.BlockSpec((8, 128), lambda s: (0, 0)),
        out_shape=jax.ShapeDtypeStruct((8, 128), jnp.float32),
        compiler_params=pltpu.CompilerParams(
            dimension_semantics=("arbitrary",),
        ),
    )(hist, dmap)
    return out[0, 0]


def kernel(points, distance_map):
    pts = points[:, :, 0, :]
    px = pts[:, :, 0]
    py = pts[:, :, 1]
    pjx = jnp.roll(px, 1, axis=1)
    pjy = jnp.roll(py, 1, axis=1)
    pts_sc = jnp.concatenate([px, py, pjx, pjy], axis=1)
    dmap = distance_map[:, :, :, 0]
    return _run(pts_sc, dmap)


# speedup vs baseline: 1.1038x; 1.0246x over previous
import functools

import jax
import jax.numpy as jnp
from jax import lax
from jax.experimental import pallas as pl
from jax.experimental.pallas import tpu as pltpu
from jax.experimental.pallas import tpu_sc as plsc

_ROWS = 256
_COLS = 256
_NEDGE = 64
_NSAM = 64
_NC = 2
_NS = 16
_NW = _NC * _NS
_RG = 16
_TROWS = 64
_NTASK = _NSAM * (_ROWS // _TROWS)
_TPW = _NTASK // _NW


def _raster_sc(pts_hbm, zeros_hbm, hist_hbm, pts_v, hist_v, sems):
    wid = lax.axis_index("s") * _NC + lax.axis_index("c")
    lane = lax.broadcasted_iota(jnp.int32, (_RG,), 0)
    lane_f = lane.astype(jnp.float32)
    ones = jnp.ones((_RG,), jnp.float32)
    n_rg = _TROWS // _RG

    def task(t, b, sem):
        g = wid * _TPW + t
        sample = g // (_ROWS // _TROWS)
        rg = g - sample * (_ROWS // _TROWS)

        @pl.when(t >= 2)
        def _drain():
            pltpu.make_async_copy(
                hist_v.at[b, :, pl.ds(0, _COLS)],
                hist_hbm.at[sample, pl.ds(rg * _TROWS, _TROWS)],
                sem).wait()

        @pl.when(rg == 0)
        def _load_pts():
            pltpu.sync_copy(pts_hbm.at[sample], pts_v)

        pltpu.sync_copy(zeros_hbm, hist_v.at[b])

        ys0 = (rg * _TROWS).astype(jnp.float32)
        yss = [lane_f + (ys0 + float(r * _RG)) for r in range(n_rg)]
        rows = [lane + r * _RG for r in range(n_rg)]
        step = jnp.full((_RG,), float(_RG), jnp.float32)
        for e in range(_NEDGE):
            px = jnp.clip(pts_v[pl.ds(e * _RG, _RG)] * 255.0, 0.0, 255.0)
            py = jnp.clip(pts_v[pl.ds((_NEDGE + e) * _RG, _RG)] * 255.0,
                          0.0, 255.0)
            pjx = jnp.clip(pts_v[pl.ds((2 * _NEDGE + e) * _RG, _RG)] * 255.0,
                           0.0, 255.0)
            pjy = jnp.clip(pts_v[pl.ds((3 * _NEDGE + e) * _RG, _RG)] * 255.0,
                           0.0, 255.0)
            dy = pjy - py
            denom = jnp.where(dy == 0.0, 1.0, dy)
            dx = pjx - px
            q = (yss[0] - py) / denom
            q_inc = step / denom
            for r in range(n_rg):
                ys = yss[r]
                cond = (py < ys) != (pjy < ys)
                xc = px + q * dx
                bins = jnp.clip(xc, 0.0, 255.0).astype(jnp.int32)
                plsc.addupdate_scatter(hist_v.at[b], [rows[r], bins], ones,
                                       mask=cond)
                if r + 1 < n_rg:
                    q = q + q_inc

        pltpu.async_copy(
            hist_v.at[b, :, pl.ds(0, _COLS)],
            hist_hbm.at[sample, pl.ds(rg * _TROWS, _TROWS)], sem)

    def pair(i, carry):
        task(i * 2, 0, sems.at[0])
        task(i * 2 + 1, 1, sems.at[1])
        return carry

    lax.fori_loop(0, _TPW // 2, pair, 0)

    for b in range(2):
        pltpu.make_async_copy(hist_v.at[b, :, pl.ds(0, _COLS)],
                              hist_hbm.at[0, pl.ds(0, _TROWS)],
                              sems.at[b]).wait()


def _dice_tc(hist_ref, dmap_ref, out_ref):
    s = pl.program_id(0)

    jj = jax.lax.broadcasted_iota(jnp.int32, (_COLS, _COLS), 0)
    xx = jax.lax.broadcasted_iota(jnp.int32, (_COLS, _COLS), 1)
    l_incl = (jj <= xx).astype(jnp.float32)

    hist = hist_ref[0]
    a = jax.lax.dot(hist, l_incl, preferred_element_type=jnp.float32)
    b = a - hist
    r_tot = a[:, _COLS - 1:_COLS]
    m_lim = r_tot - 1.0 - (r_tot - 2.0 * jnp.floor(r_tot * 0.5))
    b_odd = b - 2.0 * jnp.floor(b * 0.5)
    f_odd = (b <= m_lim).astype(jnp.float32)
    f_even = ((a > b).astype(jnp.float32)
              * ((b + 1.0) <= m_lim).astype(jnp.float32))
    filled = b_odd * f_odd + (1.0 - b_odd) * f_even

    binary = (dmap_ref[0] * 255.0 <= 127.0).astype(jnp.float32)

    inter = jnp.sum(filled * binary)
    s_true = jnp.sum(filled)
    s_pred = jnp.sum(binary)

    smooth = 1e-06
    loss = 1.0 - (2.0 * inter + smooth) / (s_true + s_pred + smooth)

    @pl.when(s == 0)
    def _init():
        out_ref[...] = jnp.zeros_like(out_ref)

    out_ref[...] += loss * (1.0 / _NSAM)


@jax.jit
def _run(pts_sc, dmap):
    zeros = jnp.zeros((_TROWS, _COLS + 1), jnp.float32)
    raster = pl.kernel(
        _raster_sc,
        out_type=jax.ShapeDtypeStruct((_NSAM, _ROWS, _COLS), jnp.float32),
        mesh=plsc.VectorSubcoreMesh(core_axis_name="c", subcore_axis_name="s",
                                    num_cores=_NC, num_subcores=_NS),
        scratch_types=[
            pltpu.VMEM((4 * _NEDGE * _RG,), jnp.float32),
            pltpu.VMEM((2, _TROWS, _COLS + 1), jnp.float32),
            pltpu.SemaphoreType.DMA((2,)),
        ],
        compiler_params=pltpu.CompilerParams(use_tc_tiling_on_sc=False,
                                             needs_layout_passes=False),
    )
    hist = raster(pts_sc, zeros)

    out = pl.pallas_call(
        _dice_tc,
        grid=(_NSAM,),
        in_specs=[
            pl.BlockSpec((1, _ROWS, _COLS), lambda s: (s, 0, 0)),
            pl.BlockSpec((1, _ROWS, _COLS), lambda s: (s, 0, 0)),
        ],
        out_specs=pl.BlockSpec((8, 128), lambda s: (0, 0)),
        out_shape=jax.ShapeDtypeStruct((8, 128), jnp.float32),
        compiler_params=pltpu.CompilerParams(
            dimension_semantics=("arbitrary",),
        ),
    )(hist, dmap)
    return out[0, 0]


def kernel(points, distance_map):
    pts = points[:, :, 0, :]
    px = pts[:, :, 0]
    py = pts[:, :, 1]
    pjx = jnp.roll(px, 1, axis=1)
    pjy = jnp.roll(py, 1, axis=1)
    pts_sc = jnp.concatenate([px, py, pjx, pjy], axis=1)
    pts_sc = jnp.repeat(pts_sc[:, :, None], _RG, axis=2).reshape(_NSAM, -1)
    dmap = distance_map[:, :, :, 0]
    return _run(pts_sc, dmap)


# speedup vs baseline: 1.2501x; 1.1326x over previous
import functools

import jax
import jax.numpy as jnp
from jax import lax
from jax.experimental import pallas as pl
from jax.experimental.pallas import tpu as pltpu
from jax.experimental.pallas import tpu_sc as plsc

_ROWS = 256
_COLS = 256
_NEDGE = 64
_NSAM = 64
_NC = 2
_NS = 16
_NW = _NC * _NS
_RG = 16
_TROWS = 64
_NTASK = _NSAM * (_ROWS // _TROWS)
_TPW = _NTASK // _NW


def _raster_sc(pts_hbm, zeros_hbm, hist_hbm, pts_v, hist_v, sems):
    wid = lax.axis_index("s") * _NC + lax.axis_index("c")
    lane = lax.broadcasted_iota(jnp.int32, (_RG,), 0)
    lane_f = lane.astype(jnp.float32)
    ones = jnp.ones((_RG,), jnp.float32)
    n_rg = _TROWS // _RG

    def task(t, b, sem):
        g = wid * _TPW + t
        sample = g // (_ROWS // _TROWS)
        rg = g - sample * (_ROWS // _TROWS)

        @pl.when(t >= 2)
        def _drain():
            pltpu.make_async_copy(
                hist_v.at[b, :, pl.ds(0, _COLS)],
                hist_hbm.at[sample, pl.ds(rg * _TROWS, _TROWS)],
                sem).wait()

        @pl.when(rg == 0)
        def _load_pts():
            pltpu.sync_copy(pts_hbm.at[sample], pts_v)

        pltpu.sync_copy(zeros_hbm, hist_v.at[b])

        ys0 = (rg * _TROWS).astype(jnp.float32)
        yss = [lane_f + (ys0 + float(r * _RG)) for r in range(n_rg)]
        rows = [lane + r * _RG for r in range(n_rg)]
        step = jnp.full((_RG,), float(_RG), jnp.float32)

        @plsc.parallel_loop(0, _NEDGE, step=1, unroll=2)
        def _edges(e):
            off = e * _RG
            px = jnp.clip(pts_v[pl.ds(off, _RG)] * 255.0, 0.0, 255.0)
            py = jnp.clip(pts_v[pl.ds(_NEDGE * _RG + off, _RG)] * 255.0,
                          0.0, 255.0)
            pjx = jnp.clip(pts_v[pl.ds(2 * _NEDGE * _RG + off, _RG)] * 255.0,
                           0.0, 255.0)
            pjy = jnp.clip(pts_v[pl.ds(3 * _NEDGE * _RG + off, _RG)] * 255.0,
                           0.0, 255.0)
            dy = pjy - py
            denom = jnp.where(dy == 0.0, 1.0, dy)
            dx = pjx - px
            q = (yss[0] - py) / denom
            q_inc = step / denom
            for r in range(n_rg):
                ys = yss[r]
                cond = (py < ys) != (pjy < ys)
                xc = px + q * dx
                bins = jnp.clip(xc, 0.0, 255.0).astype(jnp.int32)
                plsc.addupdate_scatter(hist_v.at[b], [rows[r], bins], ones,
                                       mask=cond)
                if r + 1 < n_rg:
                    q = q + q_inc

        pltpu.async_copy(
            hist_v.at[b, :, pl.ds(0, _COLS)],
            hist_hbm.at[sample, pl.ds(rg * _TROWS, _TROWS)], sem)

    def pair(i, carry):
        task(i * 2, 0, sems.at[0])
        task(i * 2 + 1, 1, sems.at[1])
        return carry

    lax.fori_loop(0, _TPW // 2, pair, 0)

    for b in range(2):
        pltpu.make_async_copy(hist_v.at[b, :, pl.ds(0, _COLS)],
                              hist_hbm.at[0, pl.ds(0, _TROWS)],
                              sems.at[b]).wait()


def _dice_tc(hist_ref, dmap_ref, out_ref):
    s = pl.program_id(0)

    jj = jax.lax.broadcasted_iota(jnp.int32, (_COLS, _COLS), 0)
    xx = jax.lax.broadcasted_iota(jnp.int32, (_COLS, _COLS), 1)
    l_incl = (jj <= xx).astype(jnp.float32)

    hist = hist_ref[0]
    a = jax.lax.dot(hist, l_incl, preferred_element_type=jnp.float32)
    b = a - hist
    r_tot = a[:, _COLS - 1:_COLS]
    m_lim = r_tot - 1.0 - (r_tot - 2.0 * jnp.floor(r_tot * 0.5))
    b_odd = b - 2.0 * jnp.floor(b * 0.5)
    f_odd = (b <= m_lim).astype(jnp.float32)
    f_even = ((a > b).astype(jnp.float32)
              * ((b + 1.0) <= m_lim).astype(jnp.float32))
    filled = b_odd * f_odd + (1.0 - b_odd) * f_even

    binary = (dmap_ref[0] * 255.0 <= 127.0).astype(jnp.float32)

    inter = jnp.sum(filled * binary)
    s_true = jnp.sum(filled)
    s_pred = jnp.sum(binary)

    smooth = 1e-06
    loss = 1.0 - (2.0 * inter + smooth) / (s_true + s_pred + smooth)

    @pl.when(s == 0)
    def _init():
        out_ref[...] = jnp.zeros_like(out_ref)

    out_ref[...] += loss * (1.0 / _NSAM)


@jax.jit
def _run(pts_sc, dmap):
    zeros = jnp.zeros((_TROWS, _COLS + 1), jnp.float32)
    raster = pl.kernel(
        _raster_sc,
        out_type=jax.ShapeDtypeStruct((_NSAM, _ROWS, _COLS), jnp.float32),
        mesh=plsc.VectorSubcoreMesh(core_axis_name="c", subcore_axis_name="s",
                                    num_cores=_NC, num_subcores=_NS),
        scratch_types=[
            pltpu.VMEM((4 * _NEDGE * _RG,), jnp.float32),
            pltpu.VMEM((2, _TROWS, _COLS + 1), jnp.float32),
            pltpu.SemaphoreType.DMA((2,)),
        ],
        compiler_params=pltpu.CompilerParams(use_tc_tiling_on_sc=False,
                                             needs_layout_passes=False),
    )
    hist = raster(pts_sc, zeros)

    out = pl.pallas_call(
        _dice_tc,
        grid=(_NSAM,),
        in_specs=[
            pl.BlockSpec((1, _ROWS, _COLS), lambda s: (s, 0, 0)),
            pl.BlockSpec((1, _ROWS, _COLS), lambda s: (s, 0, 0)),
        ],
        out_specs=pl.BlockSpec((8, 128), lambda s: (0, 0)),
        out_shape=jax.ShapeDtypeStruct((8, 128), jnp.float32),
        compiler_params=pltpu.CompilerParams(
            dimension_semantics=("arbitrary",),
        ),
    )(hist, dmap)
    return out[0, 0]


def kernel(points, distance_map):
    pts = points[:, :, 0, :]
    px = pts[:, :, 0]
    py = pts[:, :, 1]
    pjx = jnp.roll(px, 1, axis=1)
    pjy = jnp.roll(py, 1, axis=1)
    pts_sc = jnp.concatenate([px, py, pjx, pjy], axis=1)
    pts_sc = jnp.repeat(pts_sc[:, :, None], _RG, axis=2).reshape(_NSAM, -1)
    dmap = distance_map[:, :, :, 0]
    return _run(pts_sc, dmap)


# speedup vs baseline: 1.2519x; 1.0014x over previous
import functools

import jax
import jax.numpy as jnp
from jax import lax
from jax.experimental import pallas as pl
from jax.experimental.pallas import tpu as pltpu
from jax.experimental.pallas import tpu_sc as plsc

_ROWS = 256
_COLS = 256
_NEDGE = 64
_NSAM = 64
_NC = 2
_NS = 16
_NW = _NC * _NS
_RG = 16
_TROWS = 64
_NTASK = _NSAM * (_ROWS // _TROWS)
_TPW = _NTASK // _NW


def _raster_sc(pts_hbm, zeros_hbm, hist_hbm, pts_v, hist_v, sems):
    wid = lax.axis_index("s") * _NC + lax.axis_index("c")
    lane = lax.broadcasted_iota(jnp.int32, (_RG,), 0)
    lane_f = lane.astype(jnp.float32)
    ones = jnp.ones((_RG,), jnp.float32)
    n_rg = _TROWS // _RG

    def task(t, b, sem):
        g = wid * _TPW + t
        sample = g // (_ROWS // _TROWS)
        rg = g - sample * (_ROWS // _TROWS)

        @pl.when(t >= 2)
        def _drain():
            pltpu.make_async_copy(
                hist_v.at[b, :, pl.ds(0, _COLS)],
                hist_hbm.at[sample, pl.ds(rg * _TROWS, _TROWS)],
                sem).wait()

        @pl.when(rg == 0)
        def _load_pts():
            pltpu.sync_copy(pts_hbm.at[sample], pts_v)

        pltpu.sync_copy(zeros_hbm, hist_v.at[b])

        ys0 = (rg * _TROWS).astype(jnp.float32)
        yss = [lane_f + (ys0 + float(r * _RG)) for r in range(n_rg)]
        rows = [lane + r * _RG for r in range(n_rg)]
        step = jnp.full((_RG,), float(_RG), jnp.float32)

        @plsc.parallel_loop(0, _NEDGE, step=1, unroll=4)
        def _edges(e):
            off = e * _RG
            px = jnp.clip(pts_v[pl.ds(off, _RG)] * 255.0, 0.0, 255.0)
            py = jnp.clip(pts_v[pl.ds(_NEDGE * _RG + off, _RG)] * 255.0,
                          0.0, 255.0)
            pjx = jnp.clip(pts_v[pl.ds(2 * _NEDGE * _RG + off, _RG)] * 255.0,
                           0.0, 255.0)
            pjy = jnp.clip(pts_v[pl.ds(3 * _NEDGE * _RG + off, _RG)] * 255.0,
                           0.0, 255.0)
            dy = pjy - py
            denom = jnp.where(dy == 0.0, 1.0, dy)
            dx = pjx - px
            q = (yss[0] - py) / denom
            q_inc = step / denom
            for r in range(n_rg):
                ys = yss[r]
                cond = (py < ys) != (pjy < ys)
                xc = px + q * dx
                bins = jnp.clip(xc, 0.0, 255.0).astype(jnp.int32)
                plsc.addupdate_scatter(hist_v.at[b], [rows[r], bins], ones,
                                       mask=cond)
                if r + 1 < n_rg:
                    q = q + q_inc

        pltpu.async_copy(
            hist_v.at[b, :, pl.ds(0, _COLS)],
            hist_hbm.at[sample, pl.ds(rg * _TROWS, _TROWS)], sem)

    def pair(i, carry):
        task(i * 2, 0, sems.at[0])
        task(i * 2 + 1, 1, sems.at[1])
        return carry

    lax.fori_loop(0, _TPW // 2, pair, 0)

    for b in range(2):
        pltpu.make_async_copy(hist_v.at[b, :, pl.ds(0, _COLS)],
                              hist_hbm.at[0, pl.ds(0, _TROWS)],
                              sems.at[b]).wait()


def _dice_tc(hist_ref, dmap_ref, out_ref):
    s = pl.program_id(0)

    jj = jax.lax.broadcasted_iota(jnp.int32, (_COLS, _COLS), 0)
    xx = jax.lax.broadcasted_iota(jnp.int32, (_COLS, _COLS), 1)
    l_incl = (jj <= xx).astype(jnp.float32)

    hist = hist_ref[0]
    a = jax.lax.dot(hist, l_incl, preferred_element_type=jnp.float32)
    b = a - hist
    r_tot = a[:, _COLS - 1:_COLS]
    m_lim = r_tot - 1.0 - (r_tot - 2.0 * jnp.floor(r_tot * 0.5))
    b_odd = b - 2.0 * jnp.floor(b * 0.5)
    f_odd = (b <= m_lim).astype(jnp.float32)
    f_even = ((a > b).astype(jnp.float32)
              * ((b + 1.0) <= m_lim).astype(jnp.float32))
    filled = b_odd * f_odd + (1.0 - b_odd) * f_even

    binary = (dmap_ref[0] * 255.0 <= 127.0).astype(jnp.float32)

    inter = jnp.sum(filled * binary)
    s_true = jnp.sum(filled)
    s_pred = jnp.sum(binary)

    smooth = 1e-06
    loss = 1.0 - (2.0 * inter + smooth) / (s_true + s_pred + smooth)

    @pl.when(s == 0)
    def _init():
        out_ref[...] = jnp.zeros_like(out_ref)

    out_ref[...] += loss * (1.0 / _NSAM)


@jax.jit
def _run(pts_sc, dmap):
    zeros = jnp.zeros((_TROWS, _COLS + 1), jnp.float32)
    raster = pl.kernel(
        _raster_sc,
        out_type=jax.ShapeDtypeStruct((_NSAM, _ROWS, _COLS), jnp.float32),
        mesh=plsc.VectorSubcoreMesh(core_axis_name="c", subcore_axis_name="s",
                                    num_cores=_NC, num_subcores=_NS),
        scratch_types=[
            pltpu.VMEM((4 * _NEDGE * _RG,), jnp.float32),
            pltpu.VMEM((2, _TROWS, _COLS + 1), jnp.float32),
            pltpu.SemaphoreType.DMA((2,)),
        ],
        compiler_params=pltpu.CompilerParams(use_tc_tiling_on_sc=False,
                                             needs_layout_passes=False),
    )
    hist = raster(pts_sc, zeros)

    out = pl.pallas_call(
        _dice_tc,
        grid=(_NSAM,),
        in_specs=[
            pl.BlockSpec((1, _ROWS, _COLS), lambda s: (s, 0, 0)),
            pl.BlockSpec((1, _ROWS, _COLS), lambda s: (s, 0, 0)),
        ],
        out_specs=pl.BlockSpec((8, 128), lambda s: (0, 0)),
        out_shape=jax.ShapeDtypeStruct((8, 128), jnp.float32),
        compiler_params=pltpu.CompilerParams(
            dimension_semantics=("arbitrary",),
        ),
    )(hist, dmap)
    return out[0, 0]


def kernel(points, distance_map):
    pts = points[:, :, 0, :]
    px = pts[:, :, 0]
    py = pts[:, :, 1]
    pjx = jnp.roll(px, 1, axis=1)
    pjy = jnp.roll(py, 1, axis=1)
    pts_sc = jnp.concatenate([px, py, pjx, pjy], axis=1)
    pts_sc = jnp.repeat(pts_sc[:, :, None], _RG, axis=2).reshape(_NSAM, -1)
    dmap = distance_map[:, :, :, 0]
    return _run(pts_sc, dmap)


# speedup vs baseline: 1.5040x; 1.2014x over previous
import functools

import jax
import jax.numpy as jnp
from jax import lax
from jax.experimental import pallas as pl
from jax.experimental.pallas import tpu as pltpu
from jax.experimental.pallas import tpu_sc as plsc

_ROWS = 256
_COLS = 256
_NEDGE = 64
_NSAM = 64
_NC = 2
_NS = 16
_NW = _NC * _NS
_RG = 16
_TROWS = 64
_NTASK = _NSAM * (_ROWS // _TROWS)
_TPW = _NTASK // _NW
_PW = _COLS // 4


def _raster_sc(pts_hbm, zeros_hbm, hist_hbm, pts_v, hist_v, sems):
    wid = lax.axis_index("s") * _NC + lax.axis_index("c")
    lane = lax.broadcasted_iota(jnp.int32, (_RG,), 0)
    lane_f = lane.astype(jnp.float32)
    one = jnp.int32(1)
    n_rg = _TROWS // _RG

    def task(t, b, sem):
        g = wid * _TPW + t
        sample = g // (_ROWS // _TROWS)
        rg = g - sample * (_ROWS // _TROWS)

        @pl.when(t >= 2)
        def _drain():
            pltpu.make_async_copy(
                hist_v.at[b, :, pl.ds(0, _PW)],
                hist_hbm.at[sample, pl.ds(rg * _TROWS, _TROWS)],
                sem).wait()

        @pl.when(rg == 0)
        def _load_pts():
            pltpu.sync_copy(pts_hbm.at[sample], pts_v)

        pltpu.sync_copy(zeros_hbm, hist_v.at[b])

        ys0 = (rg * _TROWS).astype(jnp.float32)
        yss = [lane_f + (ys0 + float(r * _RG)) for r in range(n_rg)]
        rows = [lane + r * _RG for r in range(n_rg)]
        step = jnp.full((_RG,), float(_RG), jnp.float32)

        @plsc.parallel_loop(0, _NEDGE, step=1, unroll=4)
        def _edges(e):
            off = e * _RG
            px = jnp.clip(pts_v[pl.ds(off, _RG)] * 255.0, 0.0, 255.0)
            py = jnp.clip(pts_v[pl.ds(_NEDGE * _RG + off, _RG)] * 255.0,
                          0.0, 255.0)
            pjx = jnp.clip(pts_v[pl.ds(2 * _NEDGE * _RG + off, _RG)] * 255.0,
                           0.0, 255.0)
            pjy = jnp.clip(pts_v[pl.ds(3 * _NEDGE * _RG + off, _RG)] * 255.0,
                           0.0, 255.0)
            dy = pjy - py
            denom = jnp.where(dy == 0.0, 1.0, dy)
            dx = pjx - px
            q = (yss[0] - py) / denom
            q_inc = step / denom
            for r in range(n_rg):
                ys = yss[r]
                cond = (py < ys) != (pjy < ys)
                xc = px + q * dx
                bins = jnp.clip(xc, 0.0, 255.0).astype(jnp.int32)
                words = jax.lax.shift_right_logical(bins, 2)
                val = jax.lax.shift_left(one, jax.lax.shift_left(bins & 3, 3))
                plsc.addupdate_scatter(hist_v.at[b], [rows[r], words], val,
                                       mask=cond)
                if r + 1 < n_rg:
                    q = q + q_inc

        pltpu.async_copy(
            hist_v.at[b, :, pl.ds(0, _PW)],
            hist_hbm.at[sample, pl.ds(rg * _TROWS, _TROWS)], sem)

    def pair(i, carry):
        task(i * 2, 0, sems.at[0])
        task(i * 2 + 1, 1, sems.at[1])
        return carry

    lax.fori_loop(0, _TPW // 2, pair, 0)

    for b in range(2):
        pltpu.make_async_copy(hist_v.at[b, :, pl.ds(0, _PW)],
                              hist_hbm.at[0, pl.ds(0, _TROWS)],
                              sems.at[b]).wait()


def _dice_tc(hist_ref, dmap_ref, out_ref):
    s = pl.program_id(0)

    jj = jax.lax.broadcasted_iota(jnp.int32, (_COLS, _COLS), 0)
    xx = jax.lax.broadcasted_iota(jnp.int32, (_COLS, _COLS), 1)
    bin_of = 4 * (jj % _PW) + jj // _PW
    l_incl = (bin_of <= xx).astype(jnp.float32)
    l_strict = (bin_of < xx).astype(jnp.float32)

    hp = hist_ref[0]
    parts = [((hp >> (8 * s)) & 255).astype(jnp.float32) for s in range(4)]
    hist = jnp.concatenate(parts, axis=1)
    a = jax.lax.dot(hist, l_incl, preferred_element_type=jnp.float32)
    b = jax.lax.dot(hist, l_strict, preferred_element_type=jnp.float32)
    r_tot = a[:, _COLS - 1:_COLS]
    m_lim = r_tot - 1.0 - (r_tot - 2.0 * jnp.floor(r_tot * 0.5))
    b_odd = b - 2.0 * jnp.floor(b * 0.5)
    f_odd = (b <= m_lim).astype(jnp.float32)
    f_even = ((a > b).astype(jnp.float32)
              * ((b + 1.0) <= m_lim).astype(jnp.float32))
    filled = b_odd * f_odd + (1.0 - b_odd) * f_even

    binary = (dmap_ref[0] * 255.0 <= 127.0).astype(jnp.float32)

    inter = jnp.sum(filled * binary)
    s_true = jnp.sum(filled)
    s_pred = jnp.sum(binary)

    smooth = 1e-06
    loss = 1.0 - (2.0 * inter + smooth) / (s_true + s_pred + smooth)

    @pl.when(s == 0)
    def _init():
        out_ref[...] = jnp.zeros_like(out_ref)

    out_ref[...] += loss * (1.0 / _NSAM)


@jax.jit
def _run(pts_sc, dmap):
    zeros = jnp.zeros((_TROWS, _PW + 1), jnp.int32)
    raster = pl.kernel(
        _raster_sc,
        out_type=jax.ShapeDtypeStruct((_NSAM, _ROWS, _PW), jnp.int32),
        mesh=plsc.VectorSubcoreMesh(core_axis_name="c", subcore_axis_name="s",
                                    num_cores=_NC, num_subcores=_NS),
        scratch_types=[
            pltpu.VMEM((4 * _NEDGE * _RG,), jnp.float32),
            pltpu.VMEM((2, _TROWS, _PW + 1), jnp.int32),
            pltpu.SemaphoreType.DMA((2,)),
        ],
        compiler_params=pltpu.CompilerParams(use_tc_tiling_on_sc=False,
                                             needs_layout_passes=False),
    )
    hist = raster(pts_sc, zeros)

    out = pl.pallas_call(
        _dice_tc,
        grid=(_NSAM,),
        in_specs=[
            pl.BlockSpec((1, _ROWS, _PW), lambda s: (s, 0, 0)),
            pl.BlockSpec((1, _ROWS, _COLS), lambda s: (s, 0, 0)),
        ],
        out_specs=pl.BlockSpec((8, 128), lambda s: (0, 0)),
        out_shape=jax.ShapeDtypeStruct((8, 128), jnp.float32),
        compiler_params=pltpu.CompilerParams(
            dimension_semantics=("arbitrary",),
        ),
    )(hist, dmap)
    return out[0, 0]


def kernel(points, distance_map):
    pts = points[:, :, 0, :]
    px = pts[:, :, 0]
    py = pts[:, :, 1]
    pjx = jnp.roll(px, 1, axis=1)
    pjy = jnp.roll(py, 1, axis=1)
    pts_sc = jnp.concatenate([px, py, pjx, pjy], axis=1)
    pts_sc = jnp.repeat(pts_sc[:, :, None], _RG, axis=2).reshape(_NSAM, -1)
    dmap = distance_map[:, :, :, 0]
    return _run(pts_sc, dmap)


# speedup vs baseline: 1.5203x; 1.0108x over previous
import jax
import jax.numpy as jnp
from jax import lax
from jax.experimental import pallas as pl
from jax.experimental.pallas import tpu as pltpu
from jax.experimental.pallas import tpu_sc as plsc

_ROWS = 256
_COLS = 256
_NEDGE = 64
_NSAM = 64
_NC = 2
_NS = 16
_NW = _NC * _NS
_RG = 16
_TROWS = 64
_NTASK = _NSAM * (_ROWS // _TROWS)
_TPW = _NTASK // _NW
_PW = _COLS // 4


def _raster_sc(pts_hbm, zeros_hbm, hist_hbm, pts_v, hist_v, sems):
    wid = lax.axis_index("s") * _NC + lax.axis_index("c")
    lane = lax.broadcasted_iota(jnp.int32, (_RG,), 0)
    lane_f = lane.astype(jnp.float32)
    one = jnp.int32(1)
    n_rg = _TROWS // _RG

    def task(t, b, sem):
        g = wid * _TPW + t
        sample = g // (_ROWS // _TROWS)
        rg = g - sample * (_ROWS // _TROWS)

        @pl.when(t >= 2)
        def _drain():
            pltpu.make_async_copy(
                hist_v.at[b, :, pl.ds(0, _PW)],
                hist_hbm.at[sample, pl.ds(rg * _TROWS, _TROWS)],
                sem).wait()

        @pl.when(rg == 0)
        def _load_pts():
            pltpu.sync_copy(pts_hbm.at[sample], pts_v)

        pltpu.sync_copy(zeros_hbm, hist_v.at[b])

        ys0 = (rg * _TROWS).astype(jnp.float32)
        yss = [lane_f + (ys0 + float(r * _RG)) for r in range(n_rg)]
        rows = [lane + r * _RG for r in range(n_rg)]
        step = jnp.full((_RG,), float(_RG), jnp.float32)

        @plsc.parallel_loop(0, _NEDGE, step=1, unroll=4)
        def _edges(e):
            off = e * _RG
            px = jnp.clip(pts_v[pl.ds(off, _RG)] * 255.0, 0.0, 255.0)
            py = jnp.clip(pts_v[pl.ds(_NEDGE * _RG + off, _RG)] * 255.0,
                          0.0, 255.0)
            pjx = jnp.clip(pts_v[pl.ds(2 * _NEDGE * _RG + off, _RG)] * 255.0,
                           0.0, 255.0)
            pjy = jnp.clip(pts_v[pl.ds(3 * _NEDGE * _RG + off, _RG)] * 255.0,
                           0.0, 255.0)
            dy = pjy - py
            denom = jnp.where(dy == 0.0, 1.0, dy)
            dx = pjx - px
            q = (yss[0] - py) / denom
            q_inc = step / denom
            for r in range(n_rg):
                ys = yss[r]
                cond = (py < ys) != (pjy < ys)
                xc = px + q * dx
                bins = jnp.clip(xc, 0.0, 255.0).astype(jnp.int32)
                words = jax.lax.shift_right_logical(bins, 2)
                val = jax.lax.shift_left(one,
                                         jax.lax.shift_left(bins & 3, 3))
                plsc.addupdate_scatter(hist_v.at[b], [rows[r], words], val,
                                       mask=cond)
                if r + 1 < n_rg:
                    q = q + q_inc

        pltpu.async_copy(
            hist_v.at[b, :, pl.ds(0, _PW)],
            hist_hbm.at[sample, pl.ds(rg * _TROWS, _TROWS)], sem)

    def pair(i, carry):
        task(i * 2, 0, sems.at[0])
        task(i * 2 + 1, 1, sems.at[1])
        return carry

    lax.fori_loop(0, _TPW // 2, pair, 0)

    for b in range(2):
        pltpu.make_async_copy(hist_v.at[b, :, pl.ds(0, _PW)],
                              hist_hbm.at[0, pl.ds(0, _TROWS)],
                              sems.at[b]).wait()


def _dice_tc(hist_ref, dmap_ref, out_ref):
    s = pl.program_id(0)

    jj = jax.lax.broadcasted_iota(jnp.int32, (_COLS, _COLS), 0)
    xx = jax.lax.broadcasted_iota(jnp.int32, (_COLS, _COLS), 1)
    bin_of = 4 * (jj % _PW) + jj // _PW
    l_incl = (bin_of <= xx).astype(jnp.float32)
    l_strict = (bin_of < xx).astype(jnp.float32)

    hp = hist_ref[0]
    parts = [((hp >> (8 * s)) & 255).astype(jnp.float32) for s in range(4)]
    hist = jnp.concatenate(parts, axis=1)
    a = jax.lax.dot(hist, l_incl, preferred_element_type=jnp.float32)
    b = jax.lax.dot(hist, l_strict, preferred_element_type=jnp.float32)
    r_tot = a[:, _COLS - 1:_COLS]
    m_lim = r_tot - 1.0 - (r_tot - 2.0 * jnp.floor(r_tot * 0.5))
    b_odd = b - 2.0 * jnp.floor(b * 0.5)
    u = jnp.minimum(a, m_lim)
    filled = ((u > b).astype(jnp.float32)
              + (u == b).astype(jnp.float32) * b_odd)

    binary = (dmap_ref[0] * 255.0 <= 127.0).astype(jnp.float32)
    inter = jnp.sum(filled * binary)
    s_true = jnp.sum(filled)
    s_pred = jnp.sum(binary)

    smooth = 1e-06
    loss = 1.0 - (2.0 * inter + smooth) / (s_true + s_pred + smooth)

    @pl.when(s == 0)
    def _init():
        out_ref[...] = jnp.zeros_like(out_ref)

    out_ref[...] += loss * (1.0 / _NSAM)


@jax.jit
def _run(pts_sc, dmap):
    zeros = jnp.zeros((_TROWS, _PW + 1), jnp.int32)
    raster = pl.kernel(
        _raster_sc,
        out_type=jax.ShapeDtypeStruct((_NSAM, _ROWS, _PW), jnp.int32),
        mesh=plsc.VectorSubcoreMesh(core_axis_name="c", subcore_axis_name="s",
                                    num_cores=_NC, num_subcores=_NS),
        scratch_types=[
            pltpu.VMEM((4 * _NEDGE * _RG,), jnp.float32),
            pltpu.VMEM((2, _TROWS, _PW + 1), jnp.int32),
            pltpu.SemaphoreType.DMA((2,)),
        ],
        compiler_params=pltpu.CompilerParams(use_tc_tiling_on_sc=False,
                                             needs_layout_passes=False),
    )
    hist = raster(pts_sc, zeros)

    out = pl.pallas_call(
        _dice_tc,
        grid=(_NSAM,),
        in_specs=[
            pl.BlockSpec((1, _ROWS, _PW), lambda s: (s, 0, 0)),
            pl.BlockSpec((1, _ROWS, _COLS), lambda s: (s, 0, 0)),
        ],
        out_specs=pl.BlockSpec((8, 128), lambda s: (0, 0)),
        out_shape=jax.ShapeDtypeStruct((8, 128), jnp.float32),
        compiler_params=pltpu.CompilerParams(
            dimension_semantics=("arbitrary",),
        ),
    )(hist, dmap)
    return out[0, 0]


def kernel(points, distance_map):
    pts = points[:, :, 0, :]
    px = pts[:, :, 0]
    py = pts[:, :, 1]
    pjx = jnp.roll(px, 1, axis=1)
    pjy = jnp.roll(py, 1, axis=1)
    pts_sc = jnp.concatenate([px, py, pjx, pjy], axis=1)
    pts_sc = jnp.repeat(pts_sc[:, :, None], _RG, axis=2).reshape(_NSAM, -1)
    dmap = distance_map[:, :, :, 0]
    return _run(pts_sc, dmap)
